# baseline (speedup 1.0000x reference)
import functools

import jax
import jax.numpy as jnp
from jax import lax
from jax.experimental import pallas as pl
from jax.experimental.pallas import tpu as pltpu
from jax.experimental.pallas import tpu_sc as plsc

_B = 4
_S = 8192
_D = 1024
_NC, _NS, _L = 2, 16, 16
_NW = _NC * _NS
_ROWS = _S // _NW
_R = 8
_NJ = _ROWS // _R
_ST = 3
_NG = 10


def _sc_body(x_hbm, t_hbm, out_hbm, *rest):
    xbs = tuple(tuple(rest[st * _B + b] for b in range(_B)) for st in range(_ST))
    tbs = tuple(rest[12:15])
    sis = tuple(rest[15:18])
    sos = tuple(rest[18:21])
    sts = tuple(rest[21:24])

    wid = lax.axis_index("s") * _NC + lax.axis_index("c")
    base = wid * _ROWS

    def start_in(j, st):
        for b in range(_B):
            pltpu.async_copy(
                x_hbm.at[b, pl.ds(base + j * _R, _R), :], xbs[st][b], sis[st])

    def wait_in(st):
        for b in range(_B):
            pltpu.make_async_copy(
                x_hbm.at[0, pl.ds(base, _R), :], xbs[st][b], sis[st]).wait()

    def start_t(j, ti):
        pltpu.async_copy(
            t_hbm.at[pl.ds(base + j * _R, _R), :], tbs[ti], sts[ti])

    def wait_t(ti):
        pltpu.make_async_copy(
            t_hbm.at[pl.ds(base, _R), :], tbs[ti], sts[ti]).wait()

    def start_out(j, st):
        for b in range(_B):
            pltpu.async_copy(
                xbs[st][b], out_hbm.at[b, pl.ds(base + j * _R, _R), :],
                sos[st])

    def wait_out(st):
        for b in range(_B):
            pltpu.make_async_copy(
                xbs[st][b], out_hbm.at[0, pl.ds(base, _R), :], sos[st]).wait()

    def compute(st, ti):
        def _add(u, c):
            for uu in range(2):
                s = pl.ds((u * 2 + uu) * _L, _L)
                for r in range(_R):
                    tv = tbs[ti][r, s]
                    for b in range(_B):
                        xbs[st][b][r, s] = xbs[st][b][r, s] + tv
            return c

        lax.fori_loop(0, _D // (2 * _L), _add, 0)

    start_t(0, 0)
    start_in(0, 0)
    start_in(1, 1)

    def outer(g, carry):
        for jj in range(_ST):
            j = 3 * g + jj
            wait_in(jj)
            wait_t(jj)
            start_t(j + 1, (jj + 1) % 3)
            compute(jj, jj)
            start_out(j, jj)
            if jj == 0:
                @pl.when(g > 0)
                def _():
                    wait_out((jj + 2) % 3)
            else:
                wait_out((jj + 2) % 3)
            start_in(j + 2, (jj + 2) % 3)
        return carry

    lax.fori_loop(0, _NG, outer, 0)

    wait_in(0)
    wait_t(0)
    start_t(31, 1)
    compute(0, 0)
    start_out(30, 0)
    wait_out(2)

    wait_in(1)
    wait_t(1)
    compute(1, 1)
    start_out(31, 1)
    wait_out(0)
    wait_out(1)


@jax.jit
def _sc_add(x, table):
    mesh = plsc.VectorSubcoreMesh(core_axis_name="c", subcore_axis_name="s")
    f = functools.partial(
        pl.kernel,
        mesh=mesh,
        out_type=jax.ShapeDtypeStruct((_B, _S, _D), jnp.float32),
        scratch_types=(
            [pltpu.VMEM((_R, _D), jnp.float32)] * (_ST * _B)
            + [pltpu.VMEM((_R, _D), jnp.float32)] * _ST
            + [pltpu.SemaphoreType.DMA] * (3 * _ST)
        ),
    )(_sc_body)
    return f(x, table)


def kernel(x, table):
    return _sc_add(x, table)


# speedup vs baseline: 1.0052x; 1.0052x over previous
import functools

import jax
import jax.numpy as jnp
from jax import lax
from jax.experimental import pallas as pl
from jax.experimental.pallas import tpu as pltpu
from jax.experimental.pallas import tpu_sc as plsc

_B = 4
_S = 8192
_D = 1024
_NC, _NS, _L = 2, 16, 16
_NW = _NC * _NS
_ROWS = _S // _NW
_R = 8
_NJ = _ROWS // _R
_ST = 3
_NG = 10


def _sc_body(x_hbm, t_hbm, out_hbm, *rest):
    xbs = tuple(rest[0:3])
    tbs = tuple(rest[3:6])
    sis = tuple(rest[6:9])
    sos = tuple(rest[9:12])
    sts = tuple(rest[12:15])

    wid = lax.axis_index("s") * _NC + lax.axis_index("c")
    base = wid * _ROWS

    def start_in(j, st):
        pltpu.async_copy(
            x_hbm.at[pl.ds(0, _B), pl.ds(base + j * _R, _R), :], xbs[st],
            sis[st])

    def wait_in(st):
        pltpu.make_async_copy(
            x_hbm.at[pl.ds(0, _B), pl.ds(base, _R), :], xbs[st],
            sis[st]).wait()

    def start_t(j, ti):
        pltpu.async_copy(
            t_hbm.at[pl.ds(base + j * _R, _R), :], tbs[ti], sts[ti])

    def wait_t(ti):
        pltpu.make_async_copy(
            t_hbm.at[pl.ds(base, _R), :], tbs[ti], sts[ti]).wait()

    def start_out(j, st):
        pltpu.async_copy(
            xbs[st], out_hbm.at[pl.ds(0, _B), pl.ds(base + j * _R, _R), :],
            sos[st])

    def wait_out(st):
        pltpu.make_async_copy(
            xbs[st], out_hbm.at[pl.ds(0, _B), pl.ds(base, _R), :],
            sos[st]).wait()

    def compute(st, ti):
        def _add(u, c):
            s = pl.ds(u * _L, _L)
            for r in range(_R):
                tv = tbs[ti][r, s]
                for b in range(_B):
                    xbs[st][b, r, s] = xbs[st][b, r, s] + tv
            return c

        lax.fori_loop(0, _D // _L, _add, 0)

    start_t(0, 0)
    start_in(0, 0)
    start_in(1, 1)

    def outer(g, carry):
        for jj in range(_ST):
            j = 3 * g + jj
            wait_in(jj)
            wait_t(jj)
            start_t(j + 1, (jj + 1) % 3)
            compute(jj, jj)
            start_out(j, jj)
            if jj == 0:
                @pl.when(g > 0)
                def _():
                    wait_out((jj + 2) % 3)
            else:
                wait_out((jj + 2) % 3)
            start_in(j + 2, (jj + 2) % 3)
        return carry

    lax.fori_loop(0, _NG, outer, 0)

    wait_in(0)
    wait_t(0)
    start_t(31, 1)
    compute(0, 0)
    start_out(30, 0)
    wait_out(2)

    wait_in(1)
    wait_t(1)
    compute(1, 1)
    start_out(31, 1)
    wait_out(0)
    wait_out(1)


@jax.jit
def _sc_add(x, table):
    mesh = plsc.VectorSubcoreMesh(core_axis_name="c", subcore_axis_name="s")
    f = functools.partial(
        pl.kernel,
        mesh=mesh,
        out_type=jax.ShapeDtypeStruct((_B, _S, _D), jnp.float32),
        scratch_types=(
            [pltpu.VMEM((_B, _R, _D), jnp.float32)] * _ST
            + [pltpu.VMEM((_R, _D), jnp.float32)] * _ST
            + [pltpu.SemaphoreType.DMA] * (3 * _ST)
        ),
    )(_sc_body)
    return f(x, table)


def kernel(x, table):
    return _sc_add(x, table)
